# baseline (speedup 1.0000x reference)
import functools

import jax
import jax.numpy as jnp
from jax import lax
from jax.experimental import pallas as pl
from jax.experimental.pallas import tpu as pltpu
from jax.experimental.pallas import tpu_sc as plsc

ROWS = 128
COLS = 32768
LANES = 16
GROUP = LANES * LANES
SEG = COLS // 2
SEGS_PER_ROW = COLS // SEG
NUM_CORES = 2
NUM_SUBCORES = 16
NUM_WORKERS = NUM_CORES * NUM_SUBCORES
ROWS_PER_WORKER = ROWS // NUM_WORKERS
NSEG = ROWS_PER_WORKER * SEGS_PER_ROW

_mesh = plsc.VectorSubcoreMesh(core_axis_name="c", subcore_axis_name="s")


@functools.partial(
    pl.kernel,
    mesh=_mesh,
    compiler_params=pltpu.CompilerParams(needs_layout_passes=False),
    out_type=jax.ShapeDtypeStruct((ROWS, COLS), jnp.float32),
    scratch_types=[
        pltpu.VMEM((SEG,), jnp.float32),
        pltpu.VMEM((SEG,), jnp.float32),
        pltpu.VMEM((SEG,), jnp.float32),
        pltpu.VMEM((SEG,), jnp.float32),
        pltpu.SemaphoreType.DMA,
        pltpu.SemaphoreType.DMA,
        pltpu.SemaphoreType.DMA,
        pltpu.SemaphoreType.DMA,
        pltpu.SemaphoreType.DMA,
        pltpu.SemaphoreType.DMA,
    ],
)
def _masked_cumsum_sc(
    x_hbm, m_hbm, out_hbm, xb0, xb1, mb0, mb1, sx0, sx1, sm0, sm1, so0, so1
):
    wid = lax.axis_index("s") * NUM_CORES + lax.axis_index("c")
    base16 = lax.iota(jnp.int32, LANES) * LANES
    xb, mb = [xb0, xb1], [mb0, mb1]
    sx, sm, so = [sx0, sx1], [sm0, sm1], [so0, so1]
    row0 = wid * ROWS_PER_WORKER

    def seg_slice(s):
        return (row0 + s // SEGS_PER_ROW, pl.ds((s % SEGS_PER_ROW) * SEG, SEG))

    cx, cm, cout = {}, {}, {}
    cx[0] = pltpu.async_copy(x_hbm.at[seg_slice(0)], xb[0], sx[0])
    cm[0] = pltpu.async_copy(m_hbm.at[seg_slice(0)], mb[0], sm[0])
    carry = jnp.float32(0.0)
    for s in range(NSEG):
        p = s & 1
        cx[s].wait()
        cm[s].wait()
        xvb, mvb = xb[p], mb[p]
        if s % SEGS_PER_ROW == 0:
            carry = jnp.float32(0.0)

        def group_body(g, carry, xvb=xvb, mvb=mvb):
            goff = g * GROUP
            idx = [base16 + (goff + j) for j in range(LANES)]
            cols = [
                plsc.load_gather(xvb, [idx[j]]) * plsc.load_gather(mvb, [idx[j]])
                for j in range(LANES)
            ]
            partial = cols[0]
            sums = [partial]
            for j in range(1, LANES):
                partial = partial + cols[j]
                sums.append(partial)
            lane_tot = sums[-1]
            incl = plsc.cumsum(lane_tot)
            excl_pc = incl - lane_tot + carry
            for j in range(LANES):
                plsc.store_scatter(xvb, [idx[j]], sums[j] + excl_pc)
            return carry + jnp.sum(lane_tot)

        half = SEG // GROUP // 2
        carry = lax.fori_loop(0, half, group_body, carry)
        if s + 1 < NSEG:
            if s >= 1:
                cout[s - 1].wait()
            cx[s + 1] = pltpu.async_copy(x_hbm.at[seg_slice(s + 1)], xb[1 - p], sx[1 - p])
            cm[s + 1] = pltpu.async_copy(m_hbm.at[seg_slice(s + 1)], mb[1 - p], sm[1 - p])
        carry = lax.fori_loop(half, SEG // GROUP, group_body, carry)
        cout[s] = pltpu.async_copy(xb[p], out_hbm.at[seg_slice(s)], so[p])
    cout[NSEG - 2].wait()
    cout[NSEG - 1].wait()


def kernel(x, mask):
    return _masked_cumsum_sc(x, mask.astype(jnp.float32))


# speedup vs baseline: 1.0739x; 1.0739x over previous
import functools

import jax
import jax.numpy as jnp
from jax import lax
from jax.experimental import pallas as pl
from jax.experimental.pallas import tpu as pltpu
from jax.experimental.pallas import tpu_sc as plsc

ROWS = 128
COLS = 32768
LANES = 16
GROUP = LANES * LANES
TCB = 2048
NUM_CORES = 2
NUM_SUBCORES = 16
NUM_WORKERS = NUM_CORES * NUM_SUBCORES
ROWS_PER_WORKER = ROWS // NUM_WORKERS

_mesh = plsc.VectorSubcoreMesh(core_axis_name="c", subcore_axis_name="s")


def _pack_body(m_ref, o_ref):
    o_ref[...] = pltpu.bitcast(m_ref[...].astype(jnp.int8), jnp.int32)


_pack_mask = pl.pallas_call(
    _pack_body,
    out_shape=jax.ShapeDtypeStruct((ROWS // 4, COLS), jnp.int32),
    grid=(COLS // TCB,),
    in_specs=[pl.BlockSpec((ROWS, TCB), lambda j: (0, j))],
    out_specs=pl.BlockSpec((ROWS // 4, TCB), lambda j: (0, j)),
)


@functools.partial(
    pl.kernel,
    mesh=_mesh,
    compiler_params=pltpu.CompilerParams(needs_layout_passes=False),
    out_type=jax.ShapeDtypeStruct((ROWS, COLS), jnp.float32),
    scratch_types=[
        pltpu.VMEM((COLS,), jnp.float32),
        pltpu.VMEM((COLS,), jnp.float32),
        pltpu.VMEM((COLS,), jnp.int32),
        pltpu.SemaphoreType.DMA,
        pltpu.SemaphoreType.DMA,
        pltpu.SemaphoreType.DMA,
        pltpu.SemaphoreType.DMA,
        pltpu.SemaphoreType.DMA,
    ],
)
def _masked_cumsum_sc(x_hbm, w_hbm, out_hbm, xb0, xb1, wv, sx0, sx1, sw, so0, so1):
    wid = lax.axis_index("s") * NUM_CORES + lax.axis_index("c")
    base16 = lax.iota(jnp.int32, LANES) * LANES
    xb, sx, so = [xb0, xb1], [sx0, sx1], [so0, so1]
    row0 = wid * ROWS_PER_WORKER

    cw = pltpu.async_copy(w_hbm.at[wid], wv, sw)
    cx, cout = {}, {}
    cx[0] = pltpu.async_copy(x_hbm.at[row0], xb[0], sx[0])
    cw.wait()
    for r in range(ROWS_PER_WORKER):
        p = r & 1
        cx[r].wait()
        xvb = xb[p]

        def group_body(g, carry, xvb=xvb, r=r):
            goff = g * GROUP
            idx = [base16 + (goff + j) for j in range(LANES)]
            cols = []
            for j in range(LANES):
                xc = plsc.load_gather(xvb, [idx[j]])
                wg = plsc.load_gather(wv, [idx[j]])
                bit = ((wg >> (8 * r)) if r else wg) & 1
                cols.append(xc * bit.astype(jnp.float32))
            partial = cols[0]
            sums = [partial]
            for j in range(1, LANES):
                partial = partial + cols[j]
                sums.append(partial)
            lane_tot = sums[-1]
            incl = plsc.cumsum(lane_tot)
            excl_pc = incl - lane_tot + carry
            for j in range(LANES):
                plsc.store_scatter(xvb, [idx[j]], sums[j] + excl_pc)
            return carry + jnp.sum(lane_tot)

        half = COLS // GROUP // 2
        carry = lax.fori_loop(0, half, group_body, jnp.float32(0.0))
        if r + 1 < ROWS_PER_WORKER:
            if r >= 1:
                cout[r - 1].wait()
            cx[r + 1] = pltpu.async_copy(x_hbm.at[row0 + r + 1], xb[1 - p], sx[1 - p])
        lax.fori_loop(half, COLS // GROUP, group_body, carry)
        cout[r] = pltpu.async_copy(xb[p], out_hbm.at[row0 + r], so[p])
    cout[ROWS_PER_WORKER - 2].wait()
    cout[ROWS_PER_WORKER - 1].wait()


def kernel(x, mask):
    return _masked_cumsum_sc(x, _pack_mask(mask))


# speedup vs baseline: 1.2961x; 1.2069x over previous
import functools

import jax
import jax.numpy as jnp
from jax import lax
from jax.experimental import pallas as pl
from jax.experimental.pallas import tpu as pltpu
from jax.experimental.pallas import tpu_sc as plsc

ROWS = 128
COLS = 32768
LANES = 16
GROUP = LANES * LANES
TCB = 2048
SEG = COLS // 2
SEGS_PER_ROW = COLS // SEG
NUM_CORES = 2
NUM_SUBCORES = 16
NUM_WORKERS = NUM_CORES * NUM_SUBCORES
ROWS_PER_WORKER = ROWS // NUM_WORKERS
NSEG = ROWS_PER_WORKER * SEGS_PER_ROW

_mesh = plsc.VectorSubcoreMesh(core_axis_name="c", subcore_axis_name="s")


def _mask_body(x_ref, m_ref, o_ref):
    o_ref[...] = jnp.where(m_ref[...] != 0, x_ref[...], 0.0)


_premask = pl.pallas_call(
    _mask_body,
    out_shape=jax.ShapeDtypeStruct((ROWS, COLS), jnp.float32),
    grid=(COLS // TCB,),
    in_specs=[
        pl.BlockSpec((ROWS, TCB), lambda j: (0, j)),
        pl.BlockSpec((ROWS, TCB), lambda j: (0, j)),
    ],
    out_specs=pl.BlockSpec((ROWS, TCB), lambda j: (0, j)),
)


@functools.partial(
    pl.kernel,
    mesh=_mesh,
    compiler_params=pltpu.CompilerParams(needs_layout_passes=False),
    out_type=jax.ShapeDtypeStruct((ROWS, COLS), jnp.float32),
    scratch_types=[
        pltpu.VMEM((SEG,), jnp.float32),
        pltpu.VMEM((SEG,), jnp.float32),
        pltpu.SemaphoreType.DMA,
        pltpu.SemaphoreType.DMA,
        pltpu.SemaphoreType.DMA,
        pltpu.SemaphoreType.DMA,
    ],
)
def _cumsum_sc(x_hbm, out_hbm, xb0, xb1, sx0, sx1, so0, so1):
    wid = lax.axis_index("s") * NUM_CORES + lax.axis_index("c")
    base16 = lax.iota(jnp.int32, LANES) * LANES
    xb, sx, so = [xb0, xb1], [sx0, sx1], [so0, so1]
    row0 = wid * ROWS_PER_WORKER

    def seg_slice(s):
        return (row0 + s // SEGS_PER_ROW, pl.ds((s % SEGS_PER_ROW) * SEG, SEG))

    cx, cout = {}, {}
    cx[0] = pltpu.async_copy(x_hbm.at[seg_slice(0)], xb[0], sx[0])
    carry = jnp.float32(0.0)
    for s in range(NSEG):
        p = s & 1
        cx[s].wait()
        xvb = xb[p]
        if s % SEGS_PER_ROW == 0:
            carry = jnp.float32(0.0)

        def group_body(g, carry, xvb=xvb):
            goff = g * GROUP
            idx = [base16 + (goff + j) for j in range(LANES)]
            cols = [plsc.load_gather(xvb, [idx[j]]) for j in range(LANES)]
            partial = cols[0]
            sums = [partial]
            for j in range(1, LANES):
                partial = partial + cols[j]
                sums.append(partial)
            lane_tot = sums[-1]
            incl = plsc.cumsum(lane_tot)
            excl_pc = incl - lane_tot + carry
            for j in range(LANES):
                plsc.store_scatter(xvb, [idx[j]], sums[j] + excl_pc)
            return carry + jnp.sum(lane_tot)

        half = SEG // GROUP // 2
        carry = lax.fori_loop(0, half, group_body, carry)
        if s + 1 < NSEG:
            if s >= 1:
                cout[s - 1].wait()
            cx[s + 1] = pltpu.async_copy(x_hbm.at[seg_slice(s + 1)], xb[1 - p], sx[1 - p])
        carry = lax.fori_loop(half, SEG // GROUP, group_body, carry)
        cout[s] = pltpu.async_copy(xb[p], out_hbm.at[seg_slice(s)], so[p])
    cout[NSEG - 2].wait()
    cout[NSEG - 1].wait()


def kernel(x, mask):
    return _cumsum_sc(_premask(x, mask.astype(jnp.int8)))
